# baseline (speedup 1.0000x reference)
import functools

import jax
import jax.numpy as jnp
from jax.experimental import pallas as pl
from jax.experimental.pallas import tpu as pltpu

N = 4096
D1, D2, D3, D_IN = 128, 256, 512, 512
BMA = 256
BMB = 512
TM = 512


def _mega_kernel(z_ref, adj_ref, w4_ref, w5_ref, w6_ref,
                 zhat_ref, recon_ref,
                 adjbf_ref, buf1_ref, buf2_ref,
                 *, na, nb, nc, tj, bma, bmb, tm):
    s = pl.program_id(0)
    sa, sb, sc = na, na + nb, na + nb + nc

    @pl.when(s == 0)
    def _s1():
        z = z_ref[...].astype(jnp.bfloat16)
        w4 = w4_ref[...].astype(jnp.bfloat16)
        acc = jnp.dot(z, w4, preferred_element_type=jnp.float32)
        buf1_ref[:, :D2] = jnp.tanh(acc).astype(jnp.bfloat16)

    @pl.when(s < sa)
    def _stage_a():
        a = adj_ref[...].astype(jnp.bfloat16)
        adjbf_ref[pl.ds(s * bma, bma), :] = a
        acc = jnp.dot(a, buf1_ref[:, :D2], preferred_element_type=jnp.float32)
        w5 = w5_ref[...].astype(jnp.bfloat16)
        r = jnp.dot(acc.astype(jnp.bfloat16), w5,
                    preferred_element_type=jnp.float32)
        buf2_ref[pl.ds(s * bma, bma), :] = jnp.tanh(r).astype(jnp.bfloat16)

    @pl.when((s >= sa) & (s < sb))
    def _stage_b():
        m = s - sa
        a = adjbf_ref[pl.ds(m * bmb, bmb), :]
        acc = jnp.dot(a, buf2_ref[...], preferred_element_type=jnp.float32)
        w6 = w6_ref[...].astype(jnp.bfloat16)
        r = jnp.dot(acc.astype(jnp.bfloat16), w6,
                    preferred_element_type=jnp.float32)
        buf1_ref[pl.ds(m * bmb, bmb), :] = r.astype(jnp.bfloat16)

    @pl.when((s >= sb) & (s < sc))
    def _stage_c():
        m = s - sb
        a = adjbf_ref[pl.ds(m * bmb, bmb), :]
        acc = jnp.dot(a, buf1_ref[...], preferred_element_type=jnp.float32)
        zhat_ref[...] = acc
        buf2_ref[pl.ds(m * bmb, bmb), :] = acc.astype(jnp.bfloat16)

    @pl.when(s >= sc)
    def _stage_d():
        t = s - sc
        i = t // tj
        j = t % tj
        a = buf2_ref[pl.ds(i * tm, tm), :]
        b = buf2_ref[pl.ds(j * tm, tm), :]
        acc = jax.lax.dot_general(
            a, b, dimension_numbers=(((1,), (1,)), ((), ())),
            preferred_element_type=jnp.float32)
        recon_ref[...] = 0.5 * (1.0 + jnp.tanh(0.5 * acc))


def kernel(z_igae, adj, W4, W5, W6):
    n = N
    bma, bmb, tm = BMA, BMB, TM
    na, nb, nc = n // bma, n // bmb, n // bmb
    tj = n // tm
    sa, sb, sc = na, na + nb, na + nb + nc
    steps = sc + tj * tj

    def adj_map(s):
        return (jnp.minimum(s, na - 1), 0)

    def zhat_map(s):
        return (jnp.clip(s - sb, 0, nc - 1), 0)

    def recon_map(s):
        t = jnp.maximum(s - sc, 0)
        return (t // tj, t % tj)

    kern = functools.partial(_mega_kernel, na=na, nb=nb, nc=nc, tj=tj,
                             bma=bma, bmb=bmb, tm=tm)
    z_hat, z_hat_adj = pl.pallas_call(
        kern,
        grid=(steps,),
        in_specs=[
            pl.BlockSpec((n, D1), lambda s: (0, 0)),
            pl.BlockSpec((bma, n), adj_map),
            pl.BlockSpec((D1, D2), lambda s: (0, 0)),
            pl.BlockSpec((D2, D3), lambda s: (0, 0)),
            pl.BlockSpec((D3, D_IN), lambda s: (0, 0)),
        ],
        out_specs=[
            pl.BlockSpec((bmb, D_IN), zhat_map),
            pl.BlockSpec((tm, tm), recon_map),
        ],
        out_shape=[
            jax.ShapeDtypeStruct((n, D_IN), jnp.float32),
            jax.ShapeDtypeStruct((n, n), jnp.float32),
        ],
        scratch_shapes=[
            pltpu.VMEM((n, n), jnp.bfloat16),
            pltpu.VMEM((n, D_IN), jnp.bfloat16),
            pltpu.VMEM((n, D_IN), jnp.bfloat16),
        ],
        compiler_params=pltpu.CompilerParams(
            dimension_semantics=("arbitrary",),
        ),
    )(z_igae, adj, W4, W5, W6)
    return (z_hat, z_hat_adj)


# speedup vs baseline: 1.1796x; 1.1796x over previous
import jax
import jax.numpy as jnp
from jax import lax
from jax.experimental import pallas as pl
from jax.experimental.pallas import tpu as pltpu

N = 4096
D1, D2, D3, D_IN = 128, 256, 512, 512
PB = 512
TM = 1024


def _body(z_ref, adj_ref, w4_ref, w5_ref, w6_ref,
          zhat_ref, recon_ref,
          abuf_ref, s1_ref, s2_ref, s3_ref, zh_ref, rbuf_ref,
          in_sem, out_sem):
    np_ = N // PB
    tj = N // TM
    bf = jnp.bfloat16

    def in_copy(k, slot):
        return pltpu.make_async_copy(
            adj_ref.at[pl.ds(k * PB, PB), :],
            abuf_ref.at[slot],
            in_sem.at[slot],
        )

    in_copy(0, 0).start()
    acc = jnp.dot(z_ref[...].astype(bf), w4_ref[...].astype(bf),
                  preferred_element_type=jnp.float32)
    s1_ref[...] = jnp.tanh(acc).astype(bf)

    w5 = w5_ref[...].astype(bf)
    w6 = w6_ref[...].astype(bf)

    def make_pass(p):
        def body_fn(k, carry):
            slot = lax.rem(k, 2)
            nxt = lax.rem(k + 1, np_)
            nslot = lax.rem(k + 1, 2)
            if p < 2:
                in_copy(nxt, nslot).start()
            else:
                @pl.when(k < np_ - 1)
                def _():
                    in_copy(nxt, nslot).start()
            in_copy(k, slot).wait()
            a = abuf_ref[slot].astype(bf)
            rows = pl.ds(k * PB, PB)
            if p == 0:
                acc = jnp.dot(a, s1_ref[...],
                              preferred_element_type=jnp.float32)
                r = jnp.dot(acc.astype(bf), w5,
                            preferred_element_type=jnp.float32)
                s2_ref[rows, :] = jnp.tanh(r).astype(bf)
            elif p == 1:
                acc = jnp.dot(a, s2_ref[...],
                              preferred_element_type=jnp.float32)
                r = jnp.dot(acc.astype(bf), w6,
                            preferred_element_type=jnp.float32)
                s3_ref[rows, :] = r.astype(bf)
            else:
                acc = jnp.dot(a, s3_ref[...],
                              preferred_element_type=jnp.float32)
                zhat_ref[rows, :] = acc
                zh_ref[rows, :] = acc.astype(bf)
            return carry
        return body_fn

    for p in range(3):
        lax.fori_loop(0, np_, make_pass(p), 0, unroll=False)

    def out_copy(t, slot):
        i = t // tj
        j = lax.rem(t, tj)
        return pltpu.make_async_copy(
            rbuf_ref.at[slot],
            recon_ref.at[pl.ds(i * TM, TM), pl.ds(j * TM, TM)],
            out_sem.at[slot],
        )

    n_tiles = tj * tj

    def recon_body(t, carry):
        slot = lax.rem(t, 2)

        @pl.when(t >= 2)
        def _():
            out_copy(t - 2, slot).wait()

        i = t // tj
        j = lax.rem(t, tj)
        a = zh_ref[pl.ds(i * TM, TM), :]
        b = zh_ref[pl.ds(j * TM, TM), :]
        acc = lax.dot_general(
            a, b, dimension_numbers=(((1,), (1,)), ((), ())),
            preferred_element_type=jnp.float32)
        rbuf_ref[slot] = 0.5 * (1.0 + jnp.tanh(0.5 * acc))
        out_copy(t, slot).start()
        return carry

    lax.fori_loop(0, n_tiles, recon_body, 0, unroll=False)
    out_copy(n_tiles - 2, 0).wait()
    out_copy(n_tiles - 1, 1).wait()


def kernel(z_igae, adj, W4, W5, W6):
    z_hat, z_hat_adj = pl.pallas_call(
        _body,
        in_specs=[
            pl.BlockSpec(memory_space=pltpu.VMEM),
            pl.BlockSpec(memory_space=pl.ANY),
            pl.BlockSpec(memory_space=pltpu.VMEM),
            pl.BlockSpec(memory_space=pltpu.VMEM),
            pl.BlockSpec(memory_space=pltpu.VMEM),
        ],
        out_specs=[
            pl.BlockSpec(memory_space=pltpu.VMEM),
            pl.BlockSpec(memory_space=pl.ANY),
        ],
        out_shape=[
            jax.ShapeDtypeStruct((N, D_IN), jnp.float32),
            jax.ShapeDtypeStruct((N, N), jnp.float32),
        ],
        scratch_shapes=[
            pltpu.VMEM((2, PB, N), jnp.float32),
            pltpu.VMEM((N, D2), jnp.bfloat16),
            pltpu.VMEM((N, D3), jnp.bfloat16),
            pltpu.VMEM((N, D_IN), jnp.bfloat16),
            pltpu.VMEM((N, D_IN), jnp.bfloat16),
            pltpu.VMEM((2, TM, TM), jnp.float32),
            pltpu.SemaphoreType.DMA((2,)),
            pltpu.SemaphoreType.DMA((2,)),
        ],
    )(z_igae, adj, W4, W5, W6)
    return (z_hat, z_hat_adj)


# speedup vs baseline: 1.1897x; 1.0086x over previous
import jax
import jax.numpy as jnp
from jax import lax
from jax.experimental import pallas as pl
from jax.experimental.pallas import tpu as pltpu

N = 4096
D1, D2, D3, D_IN = 128, 256, 512, 512
PB = 512
TM = 1024


def _body(z_ref, adj_ref, w4_ref, w5_ref, w6_ref,
          zhat_ref, recon_ref,
          abuf_ref, s1_ref, s2_ref, s3_ref, zh_ref, zstage_ref, rbuf_ref,
          in_sem, zh_sem, out_sem):
    np_ = N // PB
    half = np_ // 2
    total = 3 * np_
    tj = N // TM
    bf = jnp.bfloat16

    def in_copy(g, slot):
        k = lax.rem(g, np_)
        return pltpu.make_async_copy(
            adj_ref.at[pl.ds(k * PB, PB), :],
            abuf_ref.at[slot],
            in_sem.at[slot],
        )

    in_copy(0, 0).start()
    in_copy(1, 1).start()

    acc = jnp.dot(z_ref[...].astype(bf), w4_ref[...].astype(bf),
                  preferred_element_type=jnp.float32)
    s1_ref[...] = jnp.tanh(acc).astype(bf)

    w5 = w5_ref[...].astype(bf)
    w6 = w6_ref[...].astype(bf)

    def compute_panel(p, k, a, kslot):
        rows = pl.ds(k * PB, PB)
        if p == 0:
            acc = jnp.dot(a, s1_ref[...], preferred_element_type=jnp.float32)
            r = jnp.dot(acc.astype(bf), w5, preferred_element_type=jnp.float32)
            s2_ref[rows, :] = jnp.tanh(r).astype(bf)
        elif p == 1:
            acc = jnp.dot(a, s2_ref[...], preferred_element_type=jnp.float32)
            r = jnp.dot(acc.astype(bf), w6, preferred_element_type=jnp.float32)
            s3_ref[rows, :] = r.astype(bf)
        else:
            acc = jnp.dot(a, s3_ref[...], preferred_element_type=jnp.float32)

            @pl.when(k >= 2)
            def _():
                zh_copy(k - 2, kslot).wait()

            zstage_ref[kslot] = acc
            zh_copy(k, kslot).start()
            zh_ref[rows, :] = acc.astype(bf)

    def zh_copy(m, slot):
        return pltpu.make_async_copy(
            zstage_ref.at[slot],
            zhat_ref.at[pl.ds(m * PB, PB), :],
            zh_sem.at[slot],
        )

    def make_pass(p):
        def body_fn(i, carry):
            g0 = p * np_ + 2 * i
            k0 = 2 * i

            in_copy(g0, 0).wait()
            compute_panel(p, k0, abuf_ref[0].astype(bf), 0)

            @pl.when(g0 + 2 < total)
            def _():
                in_copy(g0 + 2, 0).start()

            in_copy(g0 + 1, 1).wait()
            compute_panel(p, k0 + 1, abuf_ref[1].astype(bf), 1)

            @pl.when(g0 + 3 < total)
            def _():
                in_copy(g0 + 3, 1).start()

            return carry
        return body_fn

    for p in range(3):
        lax.fori_loop(0, half, make_pass(p), 0, unroll=False)

    zh_copy(np_ - 2, 0).wait()
    zh_copy(np_ - 1, 1).wait()

    def out_copy(t, slot):
        i = t // tj
        j = lax.rem(t, tj)
        return pltpu.make_async_copy(
            rbuf_ref.at[slot],
            recon_ref.at[pl.ds(i * TM, TM), pl.ds(j * TM, TM)],
            out_sem.at[slot],
        )

    n_tiles = tj * tj

    def recon_tile(t, slot):
        i = t // tj
        j = lax.rem(t, tj)
        a = zh_ref[pl.ds(i * TM, TM), :]
        b = zh_ref[pl.ds(j * TM, TM), :]
        acc = lax.dot_general(
            a, b, dimension_numbers=(((1,), (1,)), ((), ())),
            preferred_element_type=jnp.float32)
        rbuf_ref[slot] = 0.5 * (1.0 + jnp.tanh(0.5 * acc))
        out_copy(t, slot).start()

    def recon_body(u, carry):
        t0 = 2 * u

        @pl.when(u >= 1)
        def _():
            out_copy(t0 - 2, 0).wait()

        recon_tile(t0, 0)

        @pl.when(u >= 1)
        def _():
            out_copy(t0 - 1, 1).wait()

        recon_tile(t0 + 1, 1)
        return carry

    lax.fori_loop(0, n_tiles // 2, recon_body, 0, unroll=False)
    out_copy(n_tiles - 2, 0).wait()
    out_copy(n_tiles - 1, 1).wait()


def kernel(z_igae, adj, W4, W5, W6):
    z_hat, z_hat_adj = pl.pallas_call(
        _body,
        in_specs=[
            pl.BlockSpec(memory_space=pltpu.VMEM),
            pl.BlockSpec(memory_space=pl.ANY),
            pl.BlockSpec(memory_space=pltpu.VMEM),
            pl.BlockSpec(memory_space=pltpu.VMEM),
            pl.BlockSpec(memory_space=pltpu.VMEM),
        ],
        out_specs=[
            pl.BlockSpec(memory_space=pl.ANY),
            pl.BlockSpec(memory_space=pl.ANY),
        ],
        out_shape=[
            jax.ShapeDtypeStruct((N, D_IN), jnp.float32),
            jax.ShapeDtypeStruct((N, N), jnp.float32),
        ],
        scratch_shapes=[
            pltpu.VMEM((2, PB, N), jnp.float32),
            pltpu.VMEM((N, D2), jnp.bfloat16),
            pltpu.VMEM((N, D3), jnp.bfloat16),
            pltpu.VMEM((N, D_IN), jnp.bfloat16),
            pltpu.VMEM((N, D_IN), jnp.bfloat16),
            pltpu.VMEM((2, PB, D_IN), jnp.float32),
            pltpu.VMEM((2, TM, TM), jnp.float32),
            pltpu.SemaphoreType.DMA((2,)),
            pltpu.SemaphoreType.DMA((2,)),
            pltpu.SemaphoreType.DMA((2,)),
        ],
    )(z_igae, adj, W4, W5, W6)
    return (z_hat, z_hat_adj)
